# baseline (speedup 1.0000x reference)
import functools

import jax
import jax.numpy as jnp
from jax import lax
from jax.experimental import pallas as pl
from jax.experimental.pallas import tpu as pltpu
from jax.experimental.pallas import tpu_sc as plsc


_SC_INFO = plsc.get_sparse_core_info()
_NW = _SC_INFO.num_cores * _SC_INFO.num_subcores


def _sc_gather(table, idx):
    V, D = table.shape
    B = idx.shape[0]
    b_per_w = B // _NW
    mesh = plsc.VectorSubcoreMesh(core_axis_name="c", subcore_axis_name="s")

    @functools.partial(
        pl.kernel, mesh=mesh,
        out_type=jax.ShapeDtypeStruct((B, D), jnp.float32),
        scratch_types=[
            pltpu.VMEM((b_per_w,), jnp.int32),
            pltpu.VMEM((b_per_w, D), jnp.float32),
            pltpu.SemaphoreType.DMA,
        ],
    )
    def k(table_hbm, idx_hbm, out_hbm, idx_v, rows_v, sem):
        wid = lax.axis_index("s") * _SC_INFO.num_cores + lax.axis_index("c")
        base = wid * b_per_w
        pltpu.sync_copy(idx_hbm.at[pl.ds(base, b_per_w)], idx_v)
        pltpu.async_copy(table_hbm.at[idx_v], rows_v, sem).wait()
        pltpu.sync_copy(rows_v, out_hbm.at[pl.ds(base, b_per_w)])

    return k(table, idx)


def _layer_norm(x, g, b):
    mu = jnp.mean(x, axis=-1, keepdims=True)
    v = jnp.mean((x - mu) ** 2, axis=-1, keepdims=True)
    return (x - mu) / jnp.sqrt(v + 1e-12) * g + b


def _bert_encode_jax(ids, mask, p):
    B, S = ids.shape
    D = p['emb'].shape[1]
    x = _sc_gather(p['emb'],
                   ids.reshape(-1).astype(jnp.int32)).reshape(B, S, D)
    H = 12
    dh = D // H

    def sp(t):
        return t.reshape(B, S, H, dh).transpose(0, 2, 1, 3)

    q = sp(x @ p['Wq'] + p['bq'])
    k = sp(x @ p['Wk'] + p['bk'])
    v = sp(x @ p['Wv'] + p['bv'])
    att = q @ k.transpose(0, 1, 3, 2) / jnp.sqrt(float(dh))
    att = att + (mask[:, None, None, :] - 1.0) * 1e9
    a = jax.nn.softmax(att, axis=-1)
    o = (a @ v).transpose(0, 2, 1, 3).reshape(B, S, D) @ p['Wo'] + p['bo']
    x = _layer_norm(x + o, p['g1'], p['be1'])
    h = jax.nn.gelu(x @ p['W1'] + p['b1']) @ p['W2'] + p['b2']
    return _layer_norm(x + h, p['g2'], p['be2'])


def _bilstm_kernel(has_proj, xt_ref, wih_f_ref, whh_f_ref, bf_ref,
                   wih_b_ref, whh_b_ref, bb_ref, *rest):
    if has_proj:
        wc_ref, out_f_ref, out_b_ref, out9_ref, xwf_scr, xwb_scr = rest
    else:
        out_f_ref, out_b_ref, xwf_scr, xwb_scr = rest
    T, N, Din = xt_ref.shape
    Hh = whh_f_ref.shape[0]
    x2 = xt_ref[:].reshape(T * N, Din)

    xwf_scr[:] = jnp.dot(x2, wih_f_ref[:],
                         preferred_element_type=jnp.float32) + bf_ref[:]
    xwb_scr[:] = jnp.dot(x2, wih_b_ref[:],
                         preferred_element_type=jnp.float32) + bb_ref[:]
    whf = whh_f_ref[:]
    whb = whh_b_ref[:]

    def step(i, carry):
        h, c = carry
        tb = T - 1 - i
        gf = xwf_scr[pl.ds(i * N, N), :] + jnp.dot(
            h[:N], whf, preferred_element_type=jnp.float32)
        gb = xwb_scr[pl.ds(tb * N, N), :] + jnp.dot(
            h[N:], whb, preferred_element_type=jnp.float32)
        g = jnp.concatenate([gf, gb], axis=0)
        ig = jax.nn.sigmoid(g[:, 0 * Hh:1 * Hh])
        fg = jax.nn.sigmoid(g[:, 1 * Hh:2 * Hh])
        gg = jnp.tanh(g[:, 2 * Hh:3 * Hh])
        og = jax.nn.sigmoid(g[:, 3 * Hh:4 * Hh])
        c = fg * c + ig * gg
        h = og * jnp.tanh(c)
        out_f_ref[pl.ds(i, 1)] = h[None, :N]
        out_b_ref[pl.ds(tb, 1)] = h[None, N:]
        return (h, c)

    z = jnp.zeros((2 * N, Hh), jnp.float32)
    jax.lax.fori_loop(0, T, step, (z, z))

    if has_proj:
        enc = jnp.concatenate(
            [out_f_ref[:].reshape(T * N, Hh), out_b_ref[:].reshape(T * N, Hh)],
            axis=-1)
        out9_ref[:] = jnp.dot(enc, wc_ref[:],
                              preferred_element_type=jnp.float32).reshape(
                                  T, N, -1)


def _bilstm(x, wih_f, whh_f, b_f, wih_b, whh_b, b_b, wc=None):
    N, T, Din = x.shape
    Hh = whh_f.shape[0]
    xt = jnp.swapaxes(x, 0, 1)
    out_shape = [jax.ShapeDtypeStruct((T, N, Hh), jnp.float32)] * 2
    args = [xt, wih_f, whh_f, b_f.reshape(1, -1),
            wih_b, whh_b, b_b.reshape(1, -1)]
    if wc is not None:
        out_shape.append(jax.ShapeDtypeStruct((T, N, wc.shape[1]), jnp.float32))
        args.append(wc)
    outs = pl.pallas_call(
        functools.partial(_bilstm_kernel, wc is not None),
        out_shape=out_shape,
        scratch_shapes=[pltpu.VMEM((T * N, 4 * Hh), jnp.float32)] * 2,
    )(*args)
    if wc is not None:
        return jnp.swapaxes(outs[2], 0, 1)
    out = jnp.concatenate([outs[0], outs[1]], axis=-1)
    return jnp.swapaxes(out, 0, 1)


def _attn_kernel(att_ref, demb_ref, mask_ref, waa_ref, wad_ref,
                 ctx_ref, dist_ref):
    att = att_ref[:]
    demb = demb_ref[:]
    s = (jnp.sum(att * waa_ref[0][None, None, :], axis=-1)
         + jnp.sum(demb * wad_ref[0][None, None, :], axis=-1))
    s = s + (mask_ref[:] - 1.0) * 1e9
    m = jnp.max(s, axis=-1, keepdims=True)
    e = jnp.exp(s - m)
    a = e / jnp.sum(e, axis=-1, keepdims=True)
    ctx_ref[:] = jnp.sum(a[..., None] * att, axis=1)
    dist_ref[:] = jnp.sum(a[..., None] * demb, axis=1)


def _attn_block(attended, demb, mask, wa):
    TOK, K, C = attended.shape
    Dd = demb.shape[-1]
    waa = wa[:C, 0].reshape(1, C)
    wad = wa[C:, 0].reshape(1, Dd)
    return pl.pallas_call(
        _attn_kernel,
        out_shape=[jax.ShapeDtypeStruct((TOK, C), jnp.float32),
                   jax.ShapeDtypeStruct((TOK, Dd), jnp.float32)],
    )(attended, demb, mask, waa, wad)


def kernel(inputs, masks, transforms, context_inputs, context_masks,
           context_transforms, attn_sentence_idx, attn_word_idx, attn_dists,
           attn_mask, params):
    p = params
    ids = jnp.concatenate([inputs, context_inputs], axis=0)
    msk = jnp.concatenate([masks, context_masks], axis=0)
    last = _bert_encode_jax(ids, msk, p)
    tr = jnp.concatenate([transforms, context_transforms], axis=0)
    sentall = tr @ last

    lstm = _bilstm(sentall, p['ctx_Wih_f'], p['ctx_Whh_f'], p['ctx_b_f'],
                   p['ctx_Wih_b'], p['ctx_Whh_b'], p['ctx_b_b'])
    B = inputs.shape[0]
    sent = lstm[:B]
    ctx = lstm[B:]

    S2 = ctx.shape[1]
    flat = ctx.reshape(-1, ctx.shape[-1])
    idx = (attn_sentence_idx * S2 + attn_word_idx).reshape(-1)
    attended = _sc_gather(flat, idx.astype(jnp.int32))
    demb = p['dist_emb'][attn_dists.reshape(-1)]

    Bq, Sq, K = attn_sentence_idx.shape
    TOK = Bq * Sq
    ctx_vec, dist_vec = _attn_block(
        attended.reshape(TOK, K, -1), demb.reshape(TOK, K, -1),
        attn_mask.reshape(TOK, K), p['Wa'])

    comb = jnp.concatenate(
        [sent, ctx_vec.reshape(Bq, Sq, -1), dist_vec.reshape(Bq, Sq, -1)],
        axis=-1)
    Din = comb.shape[-1]
    Dpad = 640
    comb = jnp.pad(comb, ((0, 0), (0, 0), (0, Dpad - Din)))
    wih_f = jnp.pad(p['att_Wih_f'], ((0, Dpad - Din), (0, 0)))
    wih_b = jnp.pad(p['att_Wih_b'], ((0, Dpad - Din), (0, 0)))

    out = _bilstm(comb, wih_f, p['att_Whh_f'], p['att_b_f'],
                  wih_b, p['att_Whh_b'], p['att_b_b'],
                  wc=p['Wc'])
    return out + p['bc']


# speedup vs baseline: 1.0690x; 1.0690x over previous
import functools

import jax
import jax.numpy as jnp
from jax import lax
from jax.experimental import pallas as pl
from jax.experimental.pallas import tpu as pltpu
from jax.experimental.pallas import tpu_sc as plsc


def _sc_gather(table, idx):
    info = plsc.get_sparse_core_info()
    nw = info.num_cores * info.num_subcores
    V, D = table.shape
    B = idx.shape[0]
    b_per_w = B // nw
    mesh = plsc.VectorSubcoreMesh(core_axis_name="c", subcore_axis_name="s")

    @functools.partial(
        pl.kernel, mesh=mesh,
        out_type=jax.ShapeDtypeStruct((B, D), jnp.float32),
        scratch_types=[
            pltpu.VMEM((b_per_w,), jnp.int32),
            pltpu.VMEM((b_per_w, D), jnp.float32),
            pltpu.SemaphoreType.DMA,
        ],
    )
    def k(table_hbm, idx_hbm, out_hbm, idx_v, rows_v, sem):
        wid = lax.axis_index("s") * info.num_cores + lax.axis_index("c")
        base = wid * b_per_w
        pltpu.sync_copy(idx_hbm.at[pl.ds(base, b_per_w)], idx_v)
        pltpu.async_copy(table_hbm.at[idx_v], rows_v, sem).wait()
        pltpu.sync_copy(rows_v, out_hbm.at[pl.ds(base, b_per_w)])

    return k(table, idx)


def _layer_norm(x, g, b):
    mu = jnp.mean(x, axis=-1, keepdims=True)
    v = jnp.mean((x - mu) ** 2, axis=-1, keepdims=True)
    return (x - mu) / jnp.sqrt(v + 1e-12) * g + b


def _bert_encode_jax(ids, mask, p):
    B, S = ids.shape
    D = p['emb'].shape[1]
    x = _sc_gather(p['emb'],
                   ids.reshape(-1).astype(jnp.int32)).reshape(B, S, D)
    H = 12
    dh = D // H

    def sp(t):
        return t.reshape(B, S, H, dh).transpose(0, 2, 1, 3)

    q = sp(x @ p['Wq'] + p['bq'])
    k = sp(x @ p['Wk'] + p['bk'])
    v = sp(x @ p['Wv'] + p['bv'])
    att = q @ k.transpose(0, 1, 3, 2) / jnp.sqrt(float(dh))
    att = att + (mask[:, None, None, :] - 1.0) * 1e9
    a = jax.nn.softmax(att, axis=-1)
    o = (a @ v).transpose(0, 2, 1, 3).reshape(B, S, D) @ p['Wo'] + p['bo']
    x = _layer_norm(x + o, p['g1'], p['be1'])
    h = jax.nn.gelu(x @ p['W1'] + p['b1']) @ p['W2'] + p['b2']
    return _layer_norm(x + h, p['g2'], p['be2'])


def _lstm_loop(T, N, Hh, xwf_scr, xwb_scr, whf, whb, write_out):
    def step(i, carry):
        h, c = carry
        tb = T - 1 - i
        gf = xwf_scr[pl.ds(i * N, N), :] + jnp.dot(
            h[:N], whf, preferred_element_type=jnp.float32)
        gb = xwb_scr[pl.ds(tb * N, N), :] + jnp.dot(
            h[N:], whb, preferred_element_type=jnp.float32)
        g = jnp.concatenate([gf, gb], axis=0)
        ig = jax.nn.sigmoid(g[:, 0 * Hh:1 * Hh])
        fg = jax.nn.sigmoid(g[:, 1 * Hh:2 * Hh])
        gg = jnp.tanh(g[:, 2 * Hh:3 * Hh])
        og = jax.nn.sigmoid(g[:, 3 * Hh:4 * Hh])
        c = fg * c + ig * gg
        h = og * jnp.tanh(c)
        write_out(i, tb, h[:N], h[N:])
        return (h, c)

    z = jnp.zeros((2 * N, Hh), jnp.float32)
    jax.lax.fori_loop(0, T, step, (z, z))


def _stage_a_kernel(last_ref, tra_ref, trc_ref,
                    wihf_ref, whhf_ref, bf_ref, wihb_ref, whhb_ref, bb_ref,
                    bank_ref, sent_scr, xwf_scr, xwb_scr):
    NS, S, D = last_ref.shape
    NA = tra_ref.shape[0]
    T = tra_ref.shape[1]
    Hh = whhf_ref.shape[0]

    for n in range(NS):
        tr = tra_ref[n] if n < NA else trc_ref[n - NA]
        sent_scr[:, n:n + 1, :] = jnp.dot(
            tr, last_ref[n], preferred_element_type=jnp.float32)[:, None, :]

    x2 = sent_scr[:].reshape(T * NS, D)
    xwf_scr[:] = jnp.dot(x2, wihf_ref[:],
                         preferred_element_type=jnp.float32) + bf_ref[:]
    xwb_scr[:] = jnp.dot(x2, wihb_ref[:],
                         preferred_element_type=jnp.float32) + bb_ref[:]

    def write_out(tf, tb, hf, hb):
        bank_ref[pl.ds(tf, 1), :, 0:Hh] = hf[None]
        bank_ref[pl.ds(tb, 1), :, Hh:2 * Hh] = hb[None]

    _lstm_loop(T, NS, Hh, xwf_scr, xwb_scr, whhf_ref[:], whhb_ref[:],
               write_out)


def _stage_a(last, transforms, context_transforms, p):
    NA, T, S = transforms.shape
    NS = last.shape[0]
    D = last.shape[2]
    Hh = p['ctx_Whh_f'].shape[0]
    return pl.pallas_call(
        _stage_a_kernel,
        out_shape=jax.ShapeDtypeStruct((T, NS, 2 * Hh), jnp.float32),
        scratch_shapes=[pltpu.VMEM((T, NS, D), jnp.float32),
                        pltpu.VMEM((T * NS, 4 * Hh), jnp.float32),
                        pltpu.VMEM((T * NS, 4 * Hh), jnp.float32)],
    )(last, transforms, context_transforms,
      p['ctx_Wih_f'], p['ctx_Whh_f'], p['ctx_b_f'].reshape(1, -1),
      p['ctx_Wih_b'], p['ctx_Whh_b'], p['ctx_b_b'].reshape(1, -1))


def _stage_b_kernel(bank_ref, att_ref, dists_ref, mask_ref,
                    waa_ref, wad_ref, demb_tab_ref,
                    wihf_ref, whhf_ref, bf_ref, wihb_ref, whhb_ref, bb_ref,
                    wc_ref, out_ref,
                    comb_scr, xwf_scr, xwb_scr, of_scr, ob_scr):
    T, NS, C = bank_ref.shape
    TOK, K = mask_ref.shape
    NB = TOK // T
    ND = demb_tab_ref.shape[0]
    Dd = demb_tab_ref.shape[1]
    Hh = whhf_ref.shape[0]

    att = att_ref[:]
    dd = dists_ref[:]
    iota = lax.broadcasted_iota(jnp.int32, (TOK, K, ND), 2)
    od = (dd[..., None] == iota).astype(jnp.float32)
    demb = jnp.dot(od.reshape(TOK * K, ND), demb_tab_ref[:],
                   preferred_element_type=jnp.float32).reshape(TOK, K, Dd)

    s = (jnp.sum(att * waa_ref[0][None, None, :], axis=-1)
         + jnp.sum(demb * wad_ref[0][None, None, :], axis=-1))
    s = s + (mask_ref[:] - 1.0) * 1e9
    m = jnp.max(s, axis=-1, keepdims=True)
    e = jnp.exp(s - m)
    alpha = e / jnp.sum(e, axis=-1, keepdims=True)

    ctx_vec = jnp.sum(alpha[..., None] * att, axis=1)
    dist_vec = jnp.sum(alpha[..., None] * demb, axis=1)

    comb_scr[:, 0:C] = bank_ref[:, 0:NB, :].reshape(TOK, C)
    comb_scr[:, C:2 * C] = ctx_vec
    comb_scr[:, 2 * C:2 * C + Dd] = dist_vec

    x2 = comb_scr[:]
    xwf_scr[:] = jnp.dot(x2, wihf_ref[:],
                         preferred_element_type=jnp.float32) + bf_ref[:]
    xwb_scr[:] = jnp.dot(x2, wihb_ref[:],
                         preferred_element_type=jnp.float32) + bb_ref[:]

    def write_out(tf, tb, hf, hb):
        of_scr[pl.ds(tf * NB, NB), :] = hf
        ob_scr[pl.ds(tb * NB, NB), :] = hb

    _lstm_loop(T, NB, Hh, xwf_scr, xwb_scr, whhf_ref[:], whhb_ref[:],
               write_out)

    enc = jnp.concatenate([of_scr[:], ob_scr[:]], axis=-1)
    out_ref[:] = jnp.dot(enc, wc_ref[:],
                         preferred_element_type=jnp.float32).reshape(
                             T, NB, -1)


def _stage_b(bank3, attended, dists, mask, p):
    T, NS, C = bank3.shape
    TOK, K = mask.shape
    NB = TOK // T
    Dd = p['dist_emb'].shape[1]
    Hh = p['att_Whh_f'].shape[0]
    Dcomb = p['att_Wih_f'].shape[0]
    waa = p['Wa'][:C, 0].reshape(1, C)
    wad = p['Wa'][C:, 0].reshape(1, Dd)
    return pl.pallas_call(
        _stage_b_kernel,
        out_shape=jax.ShapeDtypeStruct((T, NB, p['Wc'].shape[1]),
                                       jnp.float32),
        scratch_shapes=[pltpu.VMEM((TOK, Dcomb), jnp.float32),
                        pltpu.VMEM((TOK, 4 * Hh), jnp.float32),
                        pltpu.VMEM((TOK, 4 * Hh), jnp.float32),
                        pltpu.VMEM((TOK, Hh), jnp.float32),
                        pltpu.VMEM((TOK, Hh), jnp.float32)],
    )(bank3, attended, dists, mask, waa, wad, p['dist_emb'],
      p['att_Wih_f'], p['att_Whh_f'], p['att_b_f'].reshape(1, -1),
      p['att_Wih_b'], p['att_Whh_b'], p['att_b_b'].reshape(1, -1),
      p['Wc'])


def kernel(inputs, masks, transforms, context_inputs, context_masks,
           context_transforms, attn_sentence_idx, attn_word_idx, attn_dists,
           attn_mask, params):
    p = params
    ids = jnp.concatenate([inputs, context_inputs], axis=0)
    msk = jnp.concatenate([masks, context_masks], axis=0)
    last = _bert_encode_jax(ids, msk, p)

    bank = _stage_a(last, transforms, context_transforms, p)

    B = inputs.shape[0]
    T = transforms.shape[1]
    sidx = jnp.swapaxes(attn_sentence_idx, 0, 1)
    widx = jnp.swapaxes(attn_word_idx, 0, 1)
    idx2 = (widx * bank.shape[1] + sidx + B).reshape(-1).astype(jnp.int32)
    C = bank.shape[2]
    attended = _sc_gather(bank.reshape(-1, C), idx2)

    K = attn_mask.shape[2]
    TOK = T * B
    dists = jnp.swapaxes(attn_dists, 0, 1).reshape(TOK, K).astype(jnp.int32)
    mask = jnp.swapaxes(attn_mask, 0, 1).reshape(TOK, K)
    out = _stage_b(bank, attended.reshape(TOK, K, C), dists, mask, p)
    return jnp.swapaxes(out, 0, 1) + p['bc']
